# Initial kernel scaffold; baseline (speedup 1.0000x reference)
import jax
import jax.numpy as jnp
from jax.experimental import pallas as pl


def kernel(ids, table):
    raise NotImplementedError("write your pallas kernel here")


# speedup vs baseline: 1.6019x; 1.6019x over previous
import functools

import jax
import jax.numpy as jnp
from jax import lax
from jax.experimental import pallas as pl
from jax.experimental.pallas import tpu as pltpu
from jax.experimental.pallas import tpu_sc as plsc

EMB = 32
MOD = 1000000
Q_MASK = -65536
R_MASK = 65535

_info = plsc.get_sparse_core_info()
_NC, _NS, _L = _info.num_cores, _info.num_subcores, _info.num_lanes
_NW = _NC * _NS

CHUNK = 128


@functools.cache
def _make(B):
    b_per_w = B // _NW
    n_chunks = b_per_w // CHUNK
    mesh = plsc.VectorSubcoreMesh(core_axis_name="c", subcore_axis_name="s")

    @functools.partial(
        pl.kernel,
        mesh=mesh,
        out_type=jax.ShapeDtypeStruct((B, EMB), jnp.float32),
        compiler_params=pltpu.CompilerParams(use_tc_tiling_on_sc=False),
        scratch_types=[
            pltpu.VMEM((CHUNK,), jnp.int32),
            pltpu.VMEM((CHUNK,), jnp.int32),
            pltpu.VMEM((CHUNK,), jnp.int32),
            pltpu.VMEM((CHUNK, EMB), jnp.float32),
            pltpu.VMEM((CHUNK, EMB), jnp.float32),
            pltpu.SemaphoreType.DMA,
        ],
    )
    def k(ids_hbm, table_hbm, out_hbm, ids_v, idxq_v, idxr_v, rq_v, rr_v, sem):
        wid = lax.axis_index("s") * _NC + lax.axis_index("c")
        base_w = wid * b_per_w

        def chunk_body(c, carry):
            base = base_w + c * CHUNK
            pltpu.sync_copy(ids_hbm.at[pl.ds(base, CHUNK)], ids_v)
            for i in range(CHUNK // _L):
                v = ids_v[pl.ds(i * _L, _L)]
                q = v & jnp.int32(Q_MASK)
                m = lax.rem(q, jnp.int32(MOD))
                m = jnp.where(m < 0, m + jnp.int32(MOD), m)
                idxq_v[pl.ds(i * _L, _L)] = m
                idxr_v[pl.ds(i * _L, _L)] = v & jnp.int32(R_MASK)
            cq = pltpu.async_copy(table_hbm.at[idxq_v], rq_v, sem)
            cr = pltpu.async_copy(table_hbm.at[idxr_v], rr_v, sem)
            cq.wait()
            cr.wait()

            def add_body(i, acc):
                for h in range(EMB // _L):
                    a = rq_v[i, pl.ds(h * _L, _L)]
                    b = rr_v[i, pl.ds(h * _L, _L)]
                    rq_v[i, pl.ds(h * _L, _L)] = a + b
                return acc

            lax.fori_loop(0, CHUNK, add_body, 0)
            pltpu.sync_copy(rq_v, out_hbm.at[pl.ds(base, CHUNK)])
            return carry

        lax.fori_loop(0, n_chunks, chunk_body, 0)

    return k


def kernel(ids, table):
    B = ids.shape[0] * ids.shape[1]
    out = _make(B)(ids.reshape(-1), table)
    return out.reshape(ids.shape + (table.shape[1],))


# speedup vs baseline: 1.9260x; 1.2023x over previous
import functools

import jax
import jax.numpy as jnp
from jax import lax
from jax.experimental import pallas as pl
from jax.experimental.pallas import tpu as pltpu
from jax.experimental.pallas import tpu_sc as plsc

EMB = 32
MOD = 1000000
Q_MASK = -65536
R_MASK = 65535

_info = plsc.get_sparse_core_info()
_NC, _NS, _L = _info.num_cores, _info.num_subcores, _info.num_lanes
_NW = _NC * _NS

CHUNK = 512
SUB = 128
NSUB = CHUNK // SUB


@functools.cache
def _make(B):
    b_per_w = B // _NW
    n_chunks = b_per_w // CHUNK
    n_pairs = n_chunks // 2
    mesh = plsc.VectorSubcoreMesh(core_axis_name="c", subcore_axis_name="s")

    @functools.partial(
        pl.kernel,
        mesh=mesh,
        out_type=jax.ShapeDtypeStruct((B, EMB), jnp.float32),
        compiler_params=pltpu.CompilerParams(use_tc_tiling_on_sc=False),
        scratch_types=[
            pltpu.VMEM((CHUNK,), jnp.int32),
            pltpu.VMEM((CHUNK,), jnp.int32),
            pltpu.VMEM((CHUNK,), jnp.int32),
            pltpu.VMEM((CHUNK, EMB), jnp.float32),
            pltpu.VMEM((CHUNK,), jnp.int32),
            pltpu.VMEM((CHUNK,), jnp.int32),
            pltpu.VMEM((CHUNK,), jnp.int32),
            pltpu.VMEM((CHUNK, EMB), jnp.float32),
            pltpu.SemaphoreType.DMA,
            pltpu.SemaphoreType.DMA,
            pltpu.SemaphoreType.DMA,
            pltpu.SemaphoreType.DMA,
        ],
    )
    def k(ids_hbm, table_hbm, out_hbm,
          ids0, idxq0, idxr0, rq0,
          ids1, idxq1, idxr1, rq1,
          semg0, semg1, semw0, semw1):
        wid = lax.axis_index("s") * _NC + lax.axis_index("c")
        base_w = wid * b_per_w
        slots = ((ids0, idxq0, idxr0, rq0, semg0, semw0),
                 (ids1, idxq1, idxr1, rq1, semg1, semw1))

        def fire_q(s, c, first):
            ids_v, idxq, idxr, rq, semg, semw = slots[s]
            base = base_w + c * CHUNK

            def drain_write():
                pltpu.make_async_copy(
                    rq, out_hbm.at[pl.ds(base, CHUNK)], semw).wait()

            if first:
                pass
            else:
                pl.when(c >= 2)(drain_write)

            pltpu.sync_copy(ids_hbm.at[pl.ds(base, CHUNK)], ids_v)

            def idx_body(i, carry):
                v = ids_v[pl.ds(i * _L, _L)]
                q = v & jnp.int32(Q_MASK)
                m = lax.rem(q, jnp.int32(MOD))
                m = jnp.where(m < 0, m + jnp.int32(MOD), m)
                idxq[pl.ds(i * _L, _L)] = m
                idxr[pl.ds(i * _L, _L)] = v & jnp.int32(R_MASK)
                return carry

            lax.fori_loop(0, CHUNK // _L, idx_body, 0)
            for j in range(NSUB):
                pltpu.async_copy(
                    table_hbm.at[idxq.at[pl.ds(j * SUB, SUB)]],
                    rq.at[pl.ds(j * SUB, SUB)], semg)

        def fire_r(s):
            ids_v, idxq, idxr, rq, semg, semw = slots[s]
            for j in range(NSUB):
                pltpu.make_async_copy(
                    table_hbm.at[idxq.at[pl.ds(j * SUB, SUB)]],
                    rq.at[pl.ds(j * SUB, SUB)], semg).wait()
            for j in range(NSUB):
                pltpu.async_copy(
                    table_hbm.at[idxr.at[pl.ds(j * SUB, SUB)]],
                    rq.at[pl.ds(j * SUB, SUB)], semg, add=True)

        def write_out(s, c):
            ids_v, idxq, idxr, rq, semg, semw = slots[s]
            base = base_w + c * CHUNK
            for j in range(NSUB):
                pltpu.make_async_copy(
                    table_hbm.at[idxr.at[pl.ds(j * SUB, SUB)]],
                    rq.at[pl.ds(j * SUB, SUB)], semg).wait()
            pltpu.async_copy(rq, out_hbm.at[pl.ds(base, CHUNK)], semw)

        fire_q(0, 0, first=True)

        def body(c2, carry):
            c0 = c2 * 2
            c1 = c0 + 1
            fire_q(1, c1, first=False)
            fire_r(0)
            write_out(0, c0)

            def refill():
                fire_q(0, c0 + 2, first=False)

            pl.when(c2 < n_pairs - 1)(refill)
            fire_r(1)
            write_out(1, c1)
            return carry

        lax.fori_loop(0, n_pairs, body, 0)

        last0 = base_w + (n_chunks - 2) * CHUNK
        last1 = base_w + (n_chunks - 1) * CHUNK
        pltpu.make_async_copy(
            rq0, out_hbm.at[pl.ds(last0, CHUNK)], semw0).wait()
        pltpu.make_async_copy(
            rq1, out_hbm.at[pl.ds(last1, CHUNK)], semw1).wait()

    return k


def kernel(ids, table):
    B = ids.shape[0] * ids.shape[1]
    out = _make(B)(ids.reshape(-1), table)
    return out.reshape(ids.shape + (table.shape[1],))


# speedup vs baseline: 3.0908x; 1.6048x over previous
import functools

import jax
import jax.numpy as jnp
from jax import lax
from jax.experimental import pallas as pl
from jax.experimental.pallas import tpu as pltpu
from jax.experimental.pallas import tpu_sc as plsc

EMB = 32
R_MASK = 65535
NQ = 15625
NQ_PAD = 16384
NR = 65536

_info = plsc.get_sparse_core_info()
_NC, _NS, _L = _info.num_cores, _info.num_subcores, _info.num_lanes
_NW = _NC * _NS

BCOL = 256
CHUNK = 512
SUB = 128
NSUB = CHUNK // SUB


NBLK = 7812
BPW = 245
TBLK = 128


@functools.cache
def _make_build():
    mesh = plsc.VectorSubcoreMesh(core_axis_name="c", subcore_axis_name="s")

    @functools.partial(
        pl.kernel,
        mesh=mesh,
        out_type=(jax.ShapeDtypeStruct((NQ_PAD * EMB,), jnp.float32),
                  jax.ShapeDtypeStruct((NR * EMB,), jnp.float32)),
        compiler_params=pltpu.CompilerParams(needs_layout_passes=False),
        scratch_types=[
            pltpu.VMEM((EMB, BCOL), jnp.float32),
            pltpu.VMEM((BCOL * EMB,), jnp.float32),
            pltpu.VMEM((EMB, TBLK), jnp.float32),
            pltpu.VMEM((EMB, TBLK), jnp.float32),
            pltpu.VMEM((2 * BPW * EMB,), jnp.float32),
            pltpu.SemaphoreType.DMA,
            pltpu.SemaphoreType.DMA,
        ],
    )
    def build(tT, last_row, qt_out, rt_out, stage, tstage, sblk0, sblk1,
              qflat, semr, semb):
        wid = lax.axis_index("s") * _NC + lax.axis_index("c")
        lanes = jnp.arange(_L, dtype=jnp.int32)

        def tcolumn(src, col, dst, off):
            cols = jnp.full((_L,), col, dtype=jnp.int32)
            a = plsc.load_gather(src, [lanes, cols])
            b = plsc.load_gather(src, [lanes + _L, cols])
            dst[pl.ds(off, _L)] = a
            dst[pl.ds(off + _L, _L)] = b

        rt_cols = NR // _NW

        def rt_chunk(i, carry):
            base = wid * rt_cols + i * BCOL
            pltpu.sync_copy(tT.at[pl.ds(0, EMB), pl.ds(base, BCOL)], stage)

            def tr(j, c2):
                tcolumn(stage, j, tstage, j * EMB)
                return c2

            lax.fori_loop(0, BCOL, tr, 0)
            pltpu.sync_copy(tstage, rt_out.at[pl.ds(base * EMB, BCOL * EMB)])
            return carry

        lax.fori_loop(0, rt_cols // BCOL, rt_chunk, 0)

        def write_last():
            pltpu.sync_copy(
                last_row,
                qflat.at[pl.ds(2 * (NBLK - (_NW - 1) * BPW) * EMB, EMB)])

        pl.when(wid == _NW - 1)(write_last)

        wb = wid * BPW
        sblks = (sblk0, sblk1)

        def fire_blk(s, b):
            def full():
                pltpu.async_copy(
                    tT.at[pl.ds(0, EMB), pl.ds(b * TBLK, TBLK)],
                    sblks[s], semb)

            pl.when(b < NBLK)(full)

        def drain_blk(s, b):
            def full():
                pltpu.make_async_copy(
                    tT.at[pl.ds(0, EMB), pl.ds(0, TBLK)], sblks[s],
                    semb).wait()

            pl.when(b < NBLK)(full)

        def extract_blk(s, b, i):
            def go():
                tcolumn(sblks[s], 0, qflat, (2 * i) * EMB)
                tcolumn(sblks[s], 64, qflat, (2 * i + 1) * EMB)

            pl.when(b < NBLK)(go)

        fire_blk(0, wb)

        def blk_pair(g, carry):
            i0 = g * 2
            i1 = i0 + 1
            fire_blk(1, wb + i1)
            drain_blk(0, wb + i0)
            extract_blk(0, wb + i0, i0)

            fire_blk(0, wb + i0 + 2)
            drain_blk(1, wb + i1)
            extract_blk(1, wb + i1, i1)
            return carry

        lax.fori_loop(0, BPW // 2, blk_pair, 0)
        ilast = BPW - 1
        drain_blk(0, wb + ilast)
        extract_blk(0, wb + ilast, ilast)

        pltpu.sync_copy(
            qflat, qt_out.at[pl.ds(2 * wb * EMB, 2 * BPW * EMB)])

    return build


@functools.cache
def _make_lookup(B):
    b_per_w = B // _NW
    n_chunks = b_per_w // CHUNK
    n_pairs = n_chunks // 2
    mesh = plsc.VectorSubcoreMesh(core_axis_name="c", subcore_axis_name="s")

    @functools.partial(
        pl.kernel,
        mesh=mesh,
        out_type=jax.ShapeDtypeStruct((B, EMB), jnp.float32),
        compiler_params=pltpu.CompilerParams(use_tc_tiling_on_sc=False),
        scratch_types=[
            pltpu.VMEM((CHUNK,), jnp.int32),
            pltpu.VMEM((CHUNK,), jnp.int32),
            pltpu.VMEM((CHUNK,), jnp.int32),
            pltpu.VMEM((CHUNK, EMB), jnp.float32),
            pltpu.VMEM((CHUNK,), jnp.int32),
            pltpu.VMEM((CHUNK,), jnp.int32),
            pltpu.VMEM((CHUNK,), jnp.int32),
            pltpu.VMEM((CHUNK, EMB), jnp.float32),
            pltpu.SemaphoreType.DMA,
            pltpu.SemaphoreType.DMA,
            pltpu.SemaphoreType.DMA,
            pltpu.SemaphoreType.DMA,
        ],
    )
    def k(ids_hbm, qt_hbm, rt_hbm, out_hbm,
          ids0, idxq0, idxr0, rq0,
          ids1, idxq1, idxr1, rq1,
          semg0, semg1, semw0, semw1):
        wid = lax.axis_index("s") * _NC + lax.axis_index("c")
        base_w = wid * b_per_w
        slots = ((ids0, idxq0, idxr0, rq0, semg0, semw0),
                 (ids1, idxq1, idxr1, rq1, semg1, semw1))

        def fire_q(s, c, first):
            ids_v, idxq, idxr, rq, semg, semw = slots[s]
            base = base_w + c * CHUNK

            def drain_write():
                pltpu.make_async_copy(
                    rq, out_hbm.at[pl.ds(base, CHUNK)], semw).wait()

            if not first:
                pl.when(c >= 2)(drain_write)

            pltpu.sync_copy(ids_hbm.at[pl.ds(base, CHUNK)], ids_v)

            def idx_body(i, carry):
                v = ids_v[pl.ds(i * _L, _L)]
                hi = lax.shift_right_arithmetic(v, 16)
                j = lax.rem(hi * 1024, jnp.int32(NQ))
                j = jnp.where(j < 0, j + jnp.int32(NQ), j)
                idxq[pl.ds(i * _L, _L)] = j
                idxr[pl.ds(i * _L, _L)] = v & jnp.int32(R_MASK)
                return carry

            lax.fori_loop(0, CHUNK // _L, idx_body, 0)
            for j in range(NSUB):
                pltpu.async_copy(
                    qt_hbm.at[idxq.at[pl.ds(j * SUB, SUB)]],
                    rq.at[pl.ds(j * SUB, SUB)], semg)

        def fire_r(s):
            ids_v, idxq, idxr, rq, semg, semw = slots[s]
            for j in range(NSUB):
                pltpu.make_async_copy(
                    qt_hbm.at[idxq.at[pl.ds(j * SUB, SUB)]],
                    rq.at[pl.ds(j * SUB, SUB)], semg).wait()
            for j in range(NSUB):
                pltpu.async_copy(
                    rt_hbm.at[idxr.at[pl.ds(j * SUB, SUB)]],
                    rq.at[pl.ds(j * SUB, SUB)], semg, add=True)

        def write_out(s, c):
            ids_v, idxq, idxr, rq, semg, semw = slots[s]
            base = base_w + c * CHUNK
            for j in range(NSUB):
                pltpu.make_async_copy(
                    rt_hbm.at[idxr.at[pl.ds(j * SUB, SUB)]],
                    rq.at[pl.ds(j * SUB, SUB)], semg).wait()
            pltpu.async_copy(rq, out_hbm.at[pl.ds(base, CHUNK)], semw)

        fire_q(0, 0, first=True)

        def body(c2, carry):
            c0 = c2 * 2
            c1 = c0 + 1
            fire_q(1, c1, first=False)
            fire_r(0)
            write_out(0, c0)

            def refill():
                fire_q(0, c0 + 2, first=False)

            pl.when(c2 < n_pairs - 1)(refill)
            fire_r(1)
            write_out(1, c1)
            return carry

        lax.fori_loop(0, n_pairs, body, 0)

        last0 = base_w + (n_chunks - 2) * CHUNK
        last1 = base_w + (n_chunks - 1) * CHUNK
        pltpu.make_async_copy(
            rq0, out_hbm.at[pl.ds(last0, CHUNK)], semw0).wait()
        pltpu.make_async_copy(
            rq1, out_hbm.at[pl.ds(last1, CHUNK)], semw1).wait()

    return k


def kernel(ids, table):
    B = ids.shape[0] * ids.shape[1]
    qt_flat, rt_flat = _make_build()(table.T, table[64 * (NQ - 1)])
    qt = qt_flat.reshape(NQ_PAD, EMB)
    rt = rt_flat.reshape(NR, EMB)
    out = _make_lookup(B)(ids.reshape(-1), qt, rt)
    return out.reshape(ids.shape + (EMB,))
